# baseline (speedup 1.0000x reference)
import jax
import jax.numpy as jnp
from jax.experimental import pallas as pl

_HID = 32
_LN_EPS = 1e-5


def _layernorm(x, g, b):
    mu = jnp.mean(x, axis=-1, keepdims=True)
    var = jnp.mean((x - mu) ** 2, axis=-1, keepdims=True)
    return (x - mu) / jnp.sqrt(var + _LN_EPS) * g + b


def _mlp(x, w1t, b1, w2t, b2, g, b):
    h = jnp.dot(x, w1t, preferred_element_type=jnp.float32) + b1
    h = h * jax.nn.sigmoid(h)
    e = jnp.dot(h, w2t, preferred_element_type=jnp.float32) + b2
    return _layernorm(e, g, b)


def _grid_body(x_ref, w1g_ref, w2g_ref, w1e_ref, w2e_ref,
               wa_ref, wb_ref, wc_ref, vec_ref, top_ref, bot_ref):
    x = x_ref[...]
    vec = vec_ref[...]
    grid_emb = _mlp(x, w1g_ref[...].T, vec[0:1, :], w2g_ref[...].T,
                    vec[1:2, :], vec[2:3, :], vec[3:4, :])
    wc = wc_ref[...]
    wfold = jnp.dot(wc, jnp.dot(wb_ref[...], wa_ref[...],
                                preferred_element_type=jnp.float32),
                    preferred_element_type=jnp.float32)
    top_ref[...] = jnp.dot(grid_emb, wfold.T, preferred_element_type=jnp.float32)
    r = _mlp(grid_emb, w1e_ref[...].T, vec[4:5, :], w2e_ref[...].T,
             vec[5:6, :], vec[6:7, :], vec[7:8, :])
    bot_ref[...] = jnp.dot(r, wc.T, preferred_element_type=jnp.float32)


def _mesh_body(x_ref, w1m_ref, w2m_ref, wa_ref, wb_ref, wc_ref, vec_ref,
               mid_ref):
    x = x_ref[...]
    vec = vec_ref[...]
    mesh_emb = _mlp(x, w1m_ref[...].T, vec[0:1, :], w2m_ref[...].T,
                    vec[1:2, :], vec[2:3, :], vec[3:4, :])
    wfold = jnp.dot(wc_ref[...], jnp.dot(wb_ref[...], wa_ref[...],
                                         preferred_element_type=jnp.float32),
                    preferred_element_type=jnp.float32)
    mid_ref[...] = jnp.dot(mesh_emb, wfold.T, preferred_element_type=jnp.float32)


def kernel(g2m_features, g2m_edge_index, grid_features, m2g_features,
           m2g_edge_index, m2m_features, mesh_static_features, m2m_edge_index,
           params):
    n_grid, grid_dim = grid_features.shape
    n_mesh, mesh_dim = mesh_static_features.shape

    pg = params["grid_embedder"]
    pm = params["mesh_embedder"]
    pe = params["encoding_grid_mlp"]
    wa = params["g2m_gnn"]["rx_node"]["W"]
    wb = params["processor"]["rx_node"]["W"]
    wc = params["m2g_gnn"]["rx_node"]["W"]

    vec_g = jnp.stack([
        pg["layers"][0]["b"], pg["layers"][1]["b"], pg["ln"]["g"], pg["ln"]["b"],
        pe["layers"][0]["b"], pe["layers"][1]["b"], pe["ln"]["g"], pe["ln"]["b"],
    ])
    vec_m = jnp.stack([
        pm["layers"][0]["b"], pm["layers"][1]["b"], pm["ln"]["g"], pm["ln"]["b"],
    ])

    blk = 5000
    n_blocks = n_grid // blk
    full = lambda shape: pl.BlockSpec(shape, lambda i: (0,) * len(shape))

    top_grid, bot = pl.pallas_call(
        _grid_body,
        grid=(n_blocks,),
        in_specs=[
            pl.BlockSpec((blk, grid_dim), lambda i: (i, 0)),
            full((_HID, grid_dim)), full((_HID, _HID)),
            full((_HID, _HID)), full((_HID, _HID)),
            full((_HID, _HID)), full((_HID, _HID)), full((_HID, _HID)),
            full((8, _HID)),
        ],
        out_specs=[
            pl.BlockSpec((blk, _HID), lambda i: (i, 0)),
            pl.BlockSpec((blk, _HID), lambda i: (i, 0)),
        ],
        out_shape=[
            jax.ShapeDtypeStruct((n_grid, _HID), jnp.float32),
            jax.ShapeDtypeStruct((n_grid, _HID), jnp.float32),
        ],
    )(grid_features, pg["layers"][0]["W"], pg["layers"][1]["W"],
      pe["layers"][0]["W"], pe["layers"][1]["W"], wa, wb, wc, vec_g)

    mid = pl.pallas_call(
        _mesh_body,
        in_specs=[
            pl.BlockSpec((n_mesh, mesh_dim), lambda: (0, 0)),
            pl.BlockSpec((_HID, mesh_dim), lambda: (0, 0)),
            pl.BlockSpec((_HID, _HID), lambda: (0, 0)),
            pl.BlockSpec((_HID, _HID), lambda: (0, 0)),
            pl.BlockSpec((_HID, _HID), lambda: (0, 0)),
            pl.BlockSpec((_HID, _HID), lambda: (0, 0)),
            pl.BlockSpec((4, _HID), lambda: (0, 0)),
        ],
        out_specs=pl.BlockSpec((n_mesh, _HID), lambda: (0, 0)),
        out_shape=jax.ShapeDtypeStruct((n_mesh, _HID), jnp.float32),
    )(mesh_static_features, pm["layers"][0]["W"], pm["layers"][1]["W"],
      wa, wb, wc, vec_m)

    return jnp.concatenate([top_grid, mid, bot], axis=0)


# speedup vs baseline: 6.5670x; 6.5670x over previous
import jax
import jax.numpy as jnp
from jax.experimental import pallas as pl

_HID = 32
_LN_EPS = 1e-5


def _mlp_t(x, w1, b1, w2, b2, g, b):
    h = jnp.dot(w1, x, preferred_element_type=jnp.float32) + b1
    h = h * jax.nn.sigmoid(h)
    e = jnp.dot(w2, h, preferred_element_type=jnp.float32) + b2
    mu = jnp.mean(e, axis=0, keepdims=True)
    var = jnp.mean((e - mu) ** 2, axis=0, keepdims=True)
    return (e - mu) / jnp.sqrt(var + _LN_EPS) * g + b


def _body(xg_ref, xm_ref, w1g_ref, w2g_ref, w1m_ref, w2m_ref, w1e_ref,
          w2e_ref, wa_ref, wb_ref, wc_ref, vg_ref, vm_ref, out_ref):
    n_grid = xg_ref.shape[1]
    n_mesh = xm_ref.shape[1]
    wc = wc_ref[...]
    wfold = jnp.dot(wc, jnp.dot(wb_ref[...], wa_ref[...],
                                preferred_element_type=jnp.float32),
                    preferred_element_type=jnp.float32)
    vg = vg_ref[...]
    vm = vm_ref[...]

    emb_g = _mlp_t(xg_ref[...], w1g_ref[...], vg[:, 0:1], w2g_ref[...],
                   vg[:, 1:2], vg[:, 2:3], vg[:, 3:4])
    out_ref[:, 0:n_grid] = jnp.dot(wfold, emb_g,
                                   preferred_element_type=jnp.float32)
    emb_m = _mlp_t(xm_ref[...], w1m_ref[...], vm[:, 0:1], w2m_ref[...],
                   vm[:, 1:2], vm[:, 2:3], vm[:, 3:4])
    out_ref[:, n_grid:n_grid + n_mesh] = jnp.dot(
        wfold, emb_m, preferred_element_type=jnp.float32)
    r = _mlp_t(emb_g, w1e_ref[...], vg[:, 4:5], w2e_ref[...],
               vg[:, 5:6], vg[:, 6:7], vg[:, 7:8])
    out_ref[:, n_grid + n_mesh:] = jnp.dot(
        wc, r, preferred_element_type=jnp.float32)


def kernel(g2m_features, g2m_edge_index, grid_features, m2g_features,
           m2g_edge_index, m2m_features, mesh_static_features, m2m_edge_index,
           params):
    n_grid, grid_dim = grid_features.shape
    n_mesh, mesh_dim = mesh_static_features.shape
    n_out = n_grid + n_mesh + n_grid

    pg = params["grid_embedder"]
    pm = params["mesh_embedder"]
    pe = params["encoding_grid_mlp"]
    wa = params["g2m_gnn"]["rx_node"]["W"]
    wb = params["processor"]["rx_node"]["W"]
    wc = params["m2g_gnn"]["rx_node"]["W"]

    vg = jnp.stack([
        pg["layers"][0]["b"], pg["layers"][1]["b"], pg["ln"]["g"], pg["ln"]["b"],
        pe["layers"][0]["b"], pe["layers"][1]["b"], pe["ln"]["g"], pe["ln"]["b"],
    ], axis=1)
    vm = jnp.stack([
        pm["layers"][0]["b"], pm["layers"][1]["b"], pm["ln"]["g"], pm["ln"]["b"],
    ], axis=1)

    out_t = pl.pallas_call(
        _body,
        out_shape=jax.ShapeDtypeStruct((_HID, n_out), jnp.float32),
    )(grid_features.T, mesh_static_features.T,
      pg["layers"][0]["W"], pg["layers"][1]["W"],
      pm["layers"][0]["W"], pm["layers"][1]["W"],
      pe["layers"][0]["W"], pe["layers"][1]["W"],
      wa, wb, wc, vg, vm)

    return out_t.T


# speedup vs baseline: 6.9484x; 1.0581x over previous
import jax
import jax.numpy as jnp
from jax.experimental import pallas as pl
from jax.experimental.pallas import tpu as pltpu

_HID = 32
_LN_EPS = 1e-5


def _ln_core(e):
    mu = jnp.mean(e, axis=0, keepdims=True)
    d = e - mu
    var = jnp.mean(d * d, axis=0, keepdims=True)
    return d * jax.lax.rsqrt(var + _LN_EPS)


def _emb_core(x, w1, b1, w2, b2):
    h = jnp.dot(w1, x, preferred_element_type=jnp.float32) + b1
    h = h * jax.nn.sigmoid(h)
    e = jnp.dot(w2, h, preferred_element_type=jnp.float32) + b2
    return _ln_core(e)


def _body(xg_ref, xm_ref, w1g_ref, w2g_ref, w1m_ref, w2m_ref, w1e_ref,
          w2e_ref, wa_ref, wb_ref, wc_ref, vg_ref, vgt_ref, vm_ref, vmt_ref,
          out_ref, s_ref, sem_a, sem_b, sem_c):
    n_grid = xg_ref.shape[1]
    n_mesh = xm_ref.shape[1]
    n_out = out_ref.shape[1]
    a_end = (n_grid // 128) * 128
    b_end = ((n_grid + n_mesh) // 128) * 128
    vg = vg_ref[...]
    vgt = vgt_ref[...]
    wc = wc_ref[...]
    dot = lambda a, b: jnp.dot(a, b, preferred_element_type=jnp.float32)

    wfold = dot(wc, dot(wb_ref[...], wa_ref[...]))
    g_g, b_g = vgt[2:3, :], vg[:, 3:4]
    g_e, b_e = vgt[6:7, :], vg[:, 7:8]
    wfold_g = wfold * g_g
    c_top = dot(wfold, b_g)
    w1e_g = w1e_ref[...] * g_g
    c1e = dot(w1e_ref[...], b_g) + vg[:, 4:5]
    wc_ge = wc * g_e
    c_bot = dot(wc, b_e)

    z_g = _emb_core(xg_ref[...], w1g_ref[...], vg[:, 0:1], w2g_ref[...],
                    vg[:, 1:2])
    s_ref[:, 0:n_grid] = dot(wfold_g, z_g) + c_top
    cp_a = pltpu.make_async_copy(s_ref.at[:, pl.ds(0, a_end)],
                                 out_ref.at[:, pl.ds(0, a_end)], sem_a)
    cp_a.start()

    vm = vm_ref[...]
    vmt = vmt_ref[...]
    g_m, b_m = vmt[2:3, :], vm[:, 3:4]
    z_m = _emb_core(xm_ref[...], w1m_ref[...], vm[:, 0:1], w2m_ref[...],
                    vm[:, 1:2])
    s_ref[:, n_grid:n_grid + n_mesh] = dot(wfold * g_m, z_m) + dot(wfold, b_m)
    cp_b = pltpu.make_async_copy(
        s_ref.at[:, pl.ds(a_end, b_end - a_end)],
        out_ref.at[:, pl.ds(a_end, b_end - a_end)], sem_b)
    cp_b.start()

    h2 = dot(w1e_g, z_g) + c1e
    h2 = h2 * jax.nn.sigmoid(h2)
    z_e = _ln_core(dot(w2e_ref[...], h2) + vg[:, 5:6])
    s_ref[:, n_grid + n_mesh:] = dot(wc_ge, z_e) + c_bot
    cp_c = pltpu.make_async_copy(
        s_ref.at[:, pl.ds(b_end, n_out - b_end)],
        out_ref.at[:, pl.ds(b_end, n_out - b_end)], sem_c)
    cp_c.start()

    cp_a.wait()
    cp_b.wait()
    cp_c.wait()


def kernel(g2m_features, g2m_edge_index, grid_features, m2g_features,
           m2g_edge_index, m2m_features, mesh_static_features, m2m_edge_index,
           params):
    n_grid, grid_dim = grid_features.shape
    n_mesh, mesh_dim = mesh_static_features.shape
    n_out = n_grid + n_mesh + n_grid

    pg = params["grid_embedder"]
    pm = params["mesh_embedder"]
    pe = params["encoding_grid_mlp"]
    wa = params["g2m_gnn"]["rx_node"]["W"]
    wb = params["processor"]["rx_node"]["W"]
    wc = params["m2g_gnn"]["rx_node"]["W"]

    gvecs = [
        pg["layers"][0]["b"], pg["layers"][1]["b"], pg["ln"]["g"], pg["ln"]["b"],
        pe["layers"][0]["b"], pe["layers"][1]["b"], pe["ln"]["g"], pe["ln"]["b"],
    ]
    mvecs = [
        pm["layers"][0]["b"], pm["layers"][1]["b"], pm["ln"]["g"], pm["ln"]["b"],
    ]
    vg = jnp.stack(gvecs, axis=1)
    vgt = jnp.stack(gvecs, axis=0)
    vm = jnp.stack(mvecs, axis=1)
    vmt = jnp.stack(mvecs, axis=0)

    out_t = pl.pallas_call(
        _body,
        out_specs=pl.BlockSpec(memory_space=pl.ANY),
        out_shape=jax.ShapeDtypeStruct((_HID, n_out), jnp.float32),
        scratch_shapes=[
            pltpu.VMEM((_HID, n_out), jnp.float32),
            pltpu.SemaphoreType.DMA,
            pltpu.SemaphoreType.DMA,
            pltpu.SemaphoreType.DMA,
        ],
    )(grid_features.T, mesh_static_features.T,
      pg["layers"][0]["W"], pg["layers"][1]["W"],
      pm["layers"][0]["W"], pm["layers"][1]["W"],
      pe["layers"][0]["W"], pe["layers"][1]["W"],
      wa, wb, wc, vg, vgt, vm, vmt)

    return out_t.T
